# baseline (speedup 1.0000x reference)
import functools
import math

import jax
import jax.numpy as jnp
from jax import lax
from jax.experimental import pallas as pl
from jax.experimental.pallas import tpu as pltpu
from jax.experimental.pallas import tpu_sc as plsc

VOCAB = 1000000
EMB_DIM = 64
SCALE = math.sqrt(EMB_DIM)

_NC = 2
_NS = 16
_NW = _NC * _NS

_NI = 4096
_NJ = 200
_VB_FULL = VOCAB // 128
_V_TAIL = VOCAB - _VB_FULL * 128
_R_ROWS = VOCAB // 2
_IB = _NI // 128
_JB = _NJ // 8
_NBLK = _JB * _IB
_BLK_PER_W = _NBLK // _NW


def _row_vecs():
    iota = lax.iota(jnp.int32, 16)
    return [iota + 16 * m for m in range(8)]


def _make_relay():
    mesh = plsc.VectorSubcoreMesh(core_axis_name="c", subcore_axis_name="s")

    @functools.partial(
        pl.kernel,
        mesh=mesh,
        out_type=jax.ShapeDtypeStruct((_R_ROWS, 128), jnp.float32),
        compiler_params=pltpu.CompilerParams(needs_layout_passes=False),
        scratch_types=[
            pltpu.VMEM((2, 64, 128), jnp.float32),
            pltpu.VMEM((2, 64, 128), jnp.float32),
            pltpu.SemaphoreType.DMA,
            pltpu.SemaphoreType.DMA,
            pltpu.SemaphoreType.DMA,
            pltpu.SemaphoreType.DMA,
        ],
    )
    def k(tt_hbm, r_hbm, pbuf, rbuf, lsem0, lsem1, ssem0, ssem1):
        wid = lax.axis_index("s") * _NC + lax.axis_index("c")
        lsems = (lsem0, lsem1)
        ssems = (ssem0, ssem1)
        rows = _row_vecs()

        def fire_load(vb, d):
            for fb in range(8):
                pltpu.async_copy(
                    tt_hbm.at[pl.ds(fb * 8, 8), pl.ds(vb * 128, 128)],
                    pbuf.at[d, pl.ds(fb * 8, 8), :],
                    lsems[d],
                )

        def wait_load(vb, d):
            for fb in range(8):
                pltpu.make_async_copy(
                    tt_hbm.at[pl.ds(fb * 8, 8), pl.ds(vb * 128, 128)],
                    pbuf.at[d, pl.ds(fb * 8, 8), :],
                    lsems[d],
                ).wait()

        def fire_store(vb, d):
            pltpu.async_copy(
                rbuf.at[d], r_hbm.at[pl.ds(vb * 64, 64)], ssems[d]
            )

        def wait_store(vb, d):
            pltpu.make_async_copy(
                rbuf.at[d], r_hbm.at[pl.ds(vb * 64, 64)], ssems[d]
            ).wait()

        def transpose_panel(d, n_pairs):
            @pl.loop(0, n_pairs)
            def pairs(c):
                for m in range(8):
                    col = jnp.broadcast_to(
                        (2 * c + (1 if m >= 4 else 0)).astype(jnp.int32), (16,)
                    )
                    v = plsc.load_gather(pbuf.at[d], [rows[m % 4], col])
                    rbuf[d, c, pl.ds(16 * m, 16)] = v

        def process(vb, d):
            @pl.when(vb + _NW < _VB_FULL)
            def _():
                fire_load(vb + _NW, 1 - d)

            wait_load(vb, d)

            @pl.when(vb >= 2 * _NW)
            def _():
                wait_store(vb - 2 * _NW, d)

            transpose_panel(d, 64)
            fire_store(vb, d)

        fire_load(wid, 0)

        @pl.loop(0, (_VB_FULL // _NW + 2) // 2)
        def outer(tp):
            vb0 = wid + (2 * tp) * _NW
            vb1 = vb0 + _NW

            @pl.when(vb0 < _VB_FULL)
            def _():
                process(vb0, 0)

            @pl.when(vb1 < _VB_FULL)
            def _():
                process(vb1, 1)

        wait_store(wid, 0)
        wait_store(wid, 1)

        @pl.when(wid == _VB_FULL % _NW)
        def _():
            for f in range(EMB_DIM):
                pltpu.async_copy(
                    tt_hbm.at[f, pl.ds(_VB_FULL * 128, _V_TAIL)],
                    pbuf.at[0, f, pl.ds(0, _V_TAIL)],
                    lsems[0],
                )
            for f in range(EMB_DIM):
                pltpu.make_async_copy(
                    tt_hbm.at[f, pl.ds(_VB_FULL * 128, _V_TAIL)],
                    pbuf.at[0, f, pl.ds(0, _V_TAIL)],
                    lsems[0],
                ).wait()
            transpose_panel(0, _V_TAIL // 2)
            pltpu.async_copy(
                rbuf.at[0, pl.ds(0, _V_TAIL // 2), :],
                r_hbm.at[pl.ds(_VB_FULL * 64, _V_TAIL // 2)],
                ssems[0],
            )
            pltpu.make_async_copy(
                rbuf.at[0, pl.ds(0, _V_TAIL // 2), :],
                r_hbm.at[pl.ds(_VB_FULL * 64, _V_TAIL // 2)],
                ssems[0],
            ).wait()

    return k


def _make_lookup():
    mesh = plsc.VectorSubcoreMesh(core_axis_name="c", subcore_axis_name="s")

    @functools.partial(
        pl.kernel,
        mesh=mesh,
        out_type=jax.ShapeDtypeStruct((_NJ, EMB_DIM, _NI), jnp.float32),
        compiler_params=pltpu.CompilerParams(needs_layout_passes=False),
        scratch_types=[
            pltpu.VMEM((2, 8, 128), jnp.int32),
            pltpu.VMEM((4, 2, 64), jnp.int32),
            pltpu.VMEM((4, 128), jnp.int32),
            pltpu.VMEM((4, 128, 128), jnp.float32),
            pltpu.VMEM((2, 64, 128), jnp.float32),
        ]
        + [pltpu.SemaphoreType.DMA] * 2
        + [pltpu.SemaphoreType.DMA] * 4
        + [pltpu.SemaphoreType.DMA] * 2,
    )
    def k(xt_hbm, r_hbm, out_hbm, xv, iv, hv, gv, pv, *sems):
        xsem = sems[0:2]
        gsem = sems[2:6]
        psem = sems[6:8]
        wid = lax.axis_index("s") * _NC + lax.axis_index("c")
        rows = _row_vecs()

        def fire_xload(t_rel, d):
            t = wid + t_rel * _NW
            jb = t // _IB
            ib = t % _IB
            pltpu.async_copy(
                xt_hbm.at[pl.ds(jb * 8, 8), pl.ds(ib * 128, 128)],
                xv.at[d],
                xsem[d],
            )

        def wait_xload(t_rel, d):
            t = wid + t_rel * _NW
            jb = t // _IB
            ib = t % _IB
            pltpu.make_async_copy(
                xt_hbm.at[pl.ds(jb * 8, 8), pl.ds(ib * 128, 128)],
                xv.at[d],
                xsem[d],
            ).wait()

        def prep_and_fire_gather(xd, j0, gd):
            for m in range(8):
                v = xv[xd, j0, pl.ds(16 * m, 16)]
                iv[gd, m // 4, pl.ds(16 * (m % 4), 16)] = v >> 1
                hv[gd, pl.ds(16 * m, 16)] = (v & 1) * 64
            for h in range(2):
                pltpu.async_copy(
                    r_hbm.at[iv.at[gd, h]],
                    gv.at[gd, pl.ds(64 * h, 64)],
                    gsem[gd],
                )

        def wait_gather(gd):
            for h in range(2):
                pltpu.make_async_copy(
                    r_hbm.at[iv.at[gd, h]],
                    gv.at[gd, pl.ds(64 * h, 64)],
                    gsem[gd],
                ).wait()

        def fire_pstore(t_rel, j0, pd):
            t = wid + t_rel * _NW
            jb = t // _IB
            ib = t % _IB
            j = jb * 8 + j0
            for fb in range(8):
                pltpu.async_copy(
                    pv.at[pd, pl.ds(fb * 8, 8), :],
                    out_hbm.at[j, pl.ds(fb * 8, 8), pl.ds(ib * 128, 128)],
                    psem[pd],
                )

        def wait_pstore(t_rel, j0, pd):
            t = wid + t_rel * _NW
            jb = t // _IB
            ib = t % _IB
            j = jb * 8 + j0
            for fb in range(8):
                pltpu.make_async_copy(
                    pv.at[pd, pl.ds(fb * 8, 8), :],
                    out_hbm.at[j, pl.ds(fb * 8, 8), pl.ds(ib * 128, 128)],
                    psem[pd],
                ).wait()

        def assemble(gd, pd):
            cols = [hv[gd, pl.ds(16 * m, 16)] for m in range(8)]

            @pl.loop(0, EMB_DIM, unroll=2)
            def feat(f):
                for m in range(8):
                    v = plsc.load_gather(gv.at[gd], [rows[m], cols[m] + f])
                    pv[pd, f, pl.ds(16 * m, 16)] = v * SCALE

        def unit(t_rel, half, j0):
            gd = j0 % 4
            pd = j0 % 2

            cf = (j0 + 3) // 8
            j0f = (j0 + 3) % 8
            t_relf = t_rel + cf
            xdf = (half + cf) % 2
            gdf = j0f % 4
            if cf == 1 and j0f == 0:
                @pl.when(t_relf < _BLK_PER_W)
                def _():
                    wait_xload(t_relf, xdf)

                    @pl.when(t_relf + 1 < _BLK_PER_W)
                    def _():
                        fire_xload(t_relf + 1, (xdf + 1) % 2)

            @pl.when(t_relf < _BLK_PER_W)
            def _():
                prep_and_fire_gather(xdf, j0f, gdf)

            wait_gather(gd)

            @pl.when(t_rel * 8 + j0 >= 2)
            def _():
                wait_pstore(t_rel, j0, pd)

            assemble(gd, pd)
            fire_pstore(t_rel, j0, pd)

        fire_xload(0, 0)
        wait_xload(0, 0)
        fire_xload(1, 1)
        for u in range(3):
            prep_and_fire_gather(0, u, u)

        @pl.loop(0, (_BLK_PER_W + 1) // 2)
        def outer(tp):
            for half in range(2):
                t_rel = 2 * tp + half

                @pl.when(t_rel < _BLK_PER_W)
                def _():
                    for j0 in range(8):
                        unit(t_rel, half, j0)

        wait_pstore(0, 0, 0)
        wait_pstore(0, 1, 1)

    return k


_relay = _make_relay()
_lookup = _make_lookup()


def kernel(x, table):
    r = table.reshape(_R_ROWS, 128)
    out_t = _lookup(x.T.astype(jnp.int32), r)
    return out_t.transpose(2, 0, 1)


# speedup vs baseline: 2.1127x; 2.1127x over previous
import functools
import math

import jax
import jax.numpy as jnp
from jax import lax
from jax.experimental import pallas as pl
from jax.experimental.pallas import tpu as pltpu
from jax.experimental.pallas import tpu_sc as plsc

VOCAB = 1000000
EMB_DIM = 64
SCALE = math.sqrt(EMB_DIM)

_NC = 2
_NS = 16
_NW = _NC * _NS

_B_TOTAL = 4096 * 200
_B_PER_W = _B_TOTAL // _NW
_CHUNK = 128
_N_CHUNKS = _B_PER_W // _CHUNK
_NBUF = 8
_LOOKAHEAD = 4
_SLICES = EMB_DIM // 16


def _make_sc_gather():
    mesh = plsc.VectorSubcoreMesh(core_axis_name="c", subcore_axis_name="s")

    @functools.partial(
        pl.kernel,
        mesh=mesh,
        out_type=jax.ShapeDtypeStruct((_B_TOTAL, 128), jnp.float32),
        compiler_params=pltpu.CompilerParams(use_tc_tiling_on_sc=False),
        scratch_types=[
            pltpu.VMEM((_N_CHUNKS, _CHUNK), jnp.int32),
            pltpu.VMEM((_NBUF, _CHUNK, EMB_DIM), jnp.float32),
        ]
        + [pltpu.SemaphoreType.DMA] * _NBUF
        + [pltpu.SemaphoreType.DMA] * _NBUF,
    )
    def k(idx_hbm, table_hbm, out_hbm, idx_v, rows_v, *sems):
        gsem = sems[:_NBUF]
        ssem = sems[_NBUF:]
        wid = lax.axis_index("s") * _NC + lax.axis_index("c")
        base = wid * _B_PER_W

        pltpu.sync_copy(idx_hbm.at[pl.ds(wid * _N_CHUNKS, _N_CHUNKS)], idx_v)

        def fire_gather(g, b):
            pltpu.async_copy(table_hbm.at[idx_v.at[g]], rows_v.at[b], gsem[b])

        def wait_gather(g, b):
            pltpu.make_async_copy(
                table_hbm.at[idx_v.at[g]], rows_v.at[b], gsem[b]
            ).wait()

        def fire_scatter(g, b):
            pltpu.async_copy(
                rows_v.at[b],
                out_hbm.at[pl.ds(base + g * _CHUNK, _CHUNK), pl.ds(0, EMB_DIM)],
                ssem[b],
            )

        def wait_scatter(g, b):
            pltpu.make_async_copy(
                rows_v.at[b],
                out_hbm.at[pl.ds(base + g * _CHUNK, _CHUNK), pl.ds(0, EMB_DIM)],
                ssem[b],
            ).wait()

        for b in range(_LOOKAHEAD):
            fire_gather(b, b)

        @pl.loop(0, _N_CHUNKS, step=_NBUF)
        def ring(G):
            for b in range(_NBUF):
                g = G + b
                wait_gather(g, b)

                @pl.loop(0, _CHUNK, unroll=4)
                def scale(r):
                    for j in range(_SLICES):
                        rows_v[b, r, pl.ds(j * 16, 16)] = (
                            rows_v[b, r, pl.ds(j * 16, 16)] * SCALE
                        )

                fire_scatter(g, b)

                bf = (b + _LOOKAHEAD) % _NBUF
                gf = g + _LOOKAHEAD

                @pl.when(gf < _N_CHUNKS)
                def _():
                    @pl.when(gf >= _NBUF)
                    def _():
                        wait_scatter(gf - _NBUF, bf)

                    fire_gather(gf, bf)

        for b in range(_NBUF):
            wait_scatter(_N_CHUNKS - _NBUF + b, b)

    return k


_sc_gather = _make_sc_gather()


def kernel(x, table):
    idx = x.reshape(_B_TOTAL // _CHUNK, _CHUNK).astype(jnp.int32)
    out = _sc_gather(idx, table)
    return out[:, :EMB_DIM].reshape(x.shape[0], x.shape[1], EMB_DIM)


# speedup vs baseline: 2.1142x; 1.0007x over previous
import functools
import math

import jax
import jax.numpy as jnp
from jax import lax
from jax.experimental import pallas as pl
from jax.experimental.pallas import tpu as pltpu
from jax.experimental.pallas import tpu_sc as plsc

VOCAB = 1000000
EMB_DIM = 64
SCALE = math.sqrt(EMB_DIM)

_NC = 2
_NS = 16
_NW = _NC * _NS

_B_TOTAL = 4096 * 200
_B_PER_W = _B_TOTAL // _NW
_CHUNK = 128
_N_CHUNKS = _B_PER_W // _CHUNK
_NBUF = 8
_LOOKAHEAD = 4
_SLICES = EMB_DIM // 16
_NI = 4096
_NJ = 200
_IBLK = _NI // 128


def _make_sc_gather():
    mesh = plsc.VectorSubcoreMesh(core_axis_name="c", subcore_axis_name="s")

    @functools.partial(
        pl.kernel,
        mesh=mesh,
        out_type=jax.ShapeDtypeStruct((_NI, _NJ, 128), jnp.float32),
        compiler_params=pltpu.CompilerParams(use_tc_tiling_on_sc=False),
        scratch_types=[
            pltpu.VMEM((_N_CHUNKS, _CHUNK), jnp.int32),
            pltpu.VMEM((_NBUF, _CHUNK, EMB_DIM), jnp.float32),
        ]
        + [pltpu.SemaphoreType.DMA] * _NBUF
        + [pltpu.SemaphoreType.DMA] * _NBUF,
    )
    def k(idx_hbm, table_hbm, out_hbm, idx_v, rows_v, *sems):
        gsem = sems[:_NBUF]
        ssem = sems[_NBUF:]
        wid = lax.axis_index("s") * _NC + lax.axis_index("c")
        base = wid * _B_PER_W

        pltpu.sync_copy(idx_hbm.at[pl.ds(wid * _N_CHUNKS, _N_CHUNKS)], idx_v)

        def fire_gather(g, b):
            pltpu.async_copy(table_hbm.at[idx_v.at[g]], rows_v.at[b], gsem[b])

        def wait_gather(g, b):
            pltpu.make_async_copy(
                table_hbm.at[idx_v.at[g]], rows_v.at[b], gsem[b]
            ).wait()

        def _dst(g):
            r = wid * _N_CHUNKS + g
            blkk = r // 8
            jr = r % 8
            jt = blkk // _IBLK
            ib = blkk % _IBLK
            j = jt * 8 + jr
            return out_hbm.at[pl.ds(ib * _CHUNK, _CHUNK), j, pl.ds(0, EMB_DIM)]

        def fire_scatter(g, b):
            pltpu.async_copy(rows_v.at[b], _dst(g), ssem[b])

        def wait_scatter(g, b):
            pltpu.make_async_copy(rows_v.at[b], _dst(g), ssem[b]).wait()

        for b in range(_LOOKAHEAD):
            fire_gather(b, b)

        @pl.loop(0, _N_CHUNKS, step=_NBUF)
        def ring(G):
            for b in range(_NBUF):
                g = G + b
                wait_gather(g, b)

                @pl.loop(0, _CHUNK, unroll=4)
                def scale(r):
                    for j in range(_SLICES):
                        rows_v[b, r, pl.ds(j * 16, 16)] = (
                            rows_v[b, r, pl.ds(j * 16, 16)] * SCALE
                        )

                fire_scatter(g, b)

                bf = (b + _LOOKAHEAD) % _NBUF
                gf = g + _LOOKAHEAD

                @pl.when(gf < _N_CHUNKS)
                def _():
                    @pl.when(gf >= _NBUF)
                    def _():
                        wait_scatter(gf - _NBUF, bf)

                    fire_gather(gf, bf)

        for b in range(_NBUF):
            wait_scatter(_N_CHUNKS - _NBUF + b, b)

    return k


_sc_gather = _make_sc_gather()


def kernel(x, table):
    idx = (
        x.T.reshape(_NJ // 8, 8, _IBLK, _CHUNK)
        .transpose(0, 2, 1, 3)
        .reshape(_B_TOTAL // _CHUNK, _CHUNK)
        .astype(jnp.int32)
    )
    out = _sc_gather(idx, table)
    return out[:, :, :EMB_DIM]
